# baseline (speedup 1.0000x reference)
import functools

import jax
import jax.numpy as jnp
from jax import lax
from jax.experimental import pallas as pl
from jax.experimental.pallas import tpu as pltpu
from jax.experimental.pallas import tpu_sc as plsc

N_TOK = 16384
N_EXP = 16
LANES = 16
NUM_CORES = 2
NUM_SUBCORES = 16
NUM_WORKERS = NUM_CORES * NUM_SUBCORES
TOK_PER_W = N_TOK // NUM_WORKERS
NCHUNK = 2
CHUNK = TOK_PER_W // NCHUNK
UNROLL = 4

_mesh = plsc.VectorSubcoreMesh(
    core_axis_name="c", subcore_axis_name="s",
    num_cores=NUM_CORES, num_subcores=NUM_SUBCORES)


@functools.partial(
    pl.kernel,
    out_type=jax.ShapeDtypeStruct((N_EXP, N_TOK), jnp.int32),
    mesh=_mesh,
    scratch_types=[
        pltpu.VMEM((2, N_EXP, CHUNK), jnp.float32),
        pltpu.VMEM((2, N_EXP, CHUNK), jnp.int32),
        pltpu.SemaphoreType.DMA,
        pltpu.SemaphoreType.DMA,
        pltpu.SemaphoreType.DMA,
    ],
)
def _threshold_kernel(st_hbm, ot_hbm, s_v, o_v, sem0, sem1, sem_out):
    wid = lax.axis_index("s") * NUM_CORES + lax.axis_index("c")
    t0 = wid * TOK_PER_W

    one = jnp.ones((LANES,), jnp.int32)
    zero = jnp.zeros((LANES,), jnp.int32)
    in_sems = (sem0, sem1)

    def tok_block(buf, t):
        h0 = None
        cnt = None
        for e in range(N_EXP):
            v = s_v[buf, e, pl.ds(t, LANES)]
            h = jnp.where(v > 0.0, one, zero)
            cnt = h if cnt is None else cnt + h
            if e == 0:
                h0 = h
            else:
                o_v[buf, e, pl.ds(t, LANES)] = h
        o_v[buf, 0, pl.ds(t, LANES)] = jnp.where(cnt == zero, one, h0)

    def compute(buf):
        def body(j, carry):
            for u in range(UNROLL):
                tok_block(buf, (j * UNROLL + u) * LANES)
            return carry
        lax.fori_loop(0, CHUNK // LANES // UNROLL, body, 0)

    def fire_in(c):
        return pltpu.async_copy(
            st_hbm.at[:, pl.ds(t0 + c * CHUNK, CHUNK)],
            s_v.at[c % 2], in_sems[c % 2])

    def fire_out(c):
        return pltpu.async_copy(
            o_v.at[c % 2], ot_hbm.at[:, pl.ds(t0 + c * CHUNK, CHUNK)],
            sem_out)

    handles_in = [fire_in(0), fire_in(1)]
    handles_out = []
    for c in range(NCHUNK):
        handles_in[c].wait()
        if c >= 2:
            handles_out[c - 2].wait()
        compute(c % 2)
        handles_out.append(fire_out(c))
        if c + 2 < NCHUNK:
            handles_in.append(fire_in(c + 2))
    handles_out[-2].wait()
    handles_out[-1].wait()


@jax.jit
def kernel(score):
    return _threshold_kernel(score.T).T


# speedup vs baseline: 1.0649x; 1.0649x over previous
import functools

import jax
import jax.numpy as jnp
from jax import lax
from jax.experimental import pallas as pl
from jax.experimental.pallas import tpu as pltpu
from jax.experimental.pallas import tpu_sc as plsc

N_TOK = 16384
N_EXP = 16
LANES = 16
NUM_CORES = 2
NUM_SUBCORES = 16
NUM_WORKERS = NUM_CORES * NUM_SUBCORES
TOK_PER_W = N_TOK // NUM_WORKERS
NCHUNK = 2
CHUNK = TOK_PER_W // NCHUNK

_mesh = plsc.VectorSubcoreMesh(
    core_axis_name="c", subcore_axis_name="s",
    num_cores=NUM_CORES, num_subcores=NUM_SUBCORES)


@functools.partial(
    pl.kernel,
    out_type=jax.ShapeDtypeStruct((N_EXP, N_TOK), jnp.int32),
    mesh=_mesh,
    scratch_types=[
        pltpu.VMEM((2, N_EXP, CHUNK), jnp.float32),
        pltpu.VMEM((2, N_EXP, CHUNK), jnp.int32),
        pltpu.SemaphoreType.DMA,
        pltpu.SemaphoreType.DMA,
        pltpu.SemaphoreType.DMA,
    ],
)
def _threshold_kernel(st_hbm, ot_hbm, s_v, o_v, sem0, sem1, sem_out):
    wid = lax.axis_index("s") * NUM_CORES + lax.axis_index("c")
    t0 = wid * TOK_PER_W

    one = jnp.ones((LANES,), jnp.int32)
    zero = jnp.zeros((LANES,), jnp.int32)
    in_sems = (sem0, sem1)

    def compute(buf):
        for j in range(CHUNK // LANES):
            t = j * LANES
            h0 = None
            cnt = None
            for e in range(N_EXP):
                v = s_v[buf, e, pl.ds(t, LANES)]
                h = jnp.where(v > 0.0, one, zero)
                cnt = h if cnt is None else cnt + h
                if e == 0:
                    h0 = h
                else:
                    o_v[buf, e, pl.ds(t, LANES)] = h
            o_v[buf, 0, pl.ds(t, LANES)] = jnp.where(cnt == zero, one, h0)

    def fire_in(c):
        return pltpu.async_copy(
            st_hbm.at[:, pl.ds(t0 + c * CHUNK, CHUNK)],
            s_v.at[c % 2], in_sems[c % 2])

    def fire_out(c):
        return pltpu.async_copy(
            o_v.at[c % 2], ot_hbm.at[:, pl.ds(t0 + c * CHUNK, CHUNK)],
            sem_out)

    handles_in = [fire_in(0), fire_in(1)]
    handles_out = []
    for c in range(NCHUNK):
        handles_in[c].wait()
        if c >= 2:
            handles_out[c - 2].wait()
        compute(c % 2)
        handles_out.append(fire_out(c))
        if c + 2 < NCHUNK:
            handles_in.append(fire_in(c + 2))
    handles_out[-2].wait()
    handles_out[-1].wait()


@jax.jit
def kernel(score):
    return _threshold_kernel(score.T).T
